# baseline (speedup 1.0000x reference)
import numpy as np
import jax
import jax.numpy as jnp
from jax import lax
from jax.experimental import pallas as pl
from jax.experimental.pallas import tpu as pltpu

K = 10
B = 64
N = 8192
_UMIN = np.float32(1e-6)
_UMAX = np.float32(1.0 - 1e-6)
_USCALE = np.float32(_UMAX - _UMIN)


def _threefry2x32(k1, k2, x0, x1):
    rotations = ((13, 15, 26, 6), (17, 29, 16, 24))
    ks0 = k1
    ks1 = k2
    ks2 = k1 ^ k2 ^ np.uint32(0x1BD11BDA)
    ks = (ks0, ks1, ks2)
    v0 = x0 + ks0
    v1 = x1 + ks1
    for i in range(5):
        for r in rotations[i % 2]:
            v0 = v0 + v1
            v1 = (v1 << np.uint32(r)) | (v1 >> np.uint32(32 - r))
            v1 = v0 ^ v1
        v0 = v0 + ks[(i + 1) % 3]
        v1 = v1 + ks[(i + 2) % 3] + np.uint32(i + 1)
    return v0, v1


def _bits_to_uniform(bits):
    fb = (bits >> np.uint32(9)) | np.uint32(0x3F800000)
    f1 = lax.bitcast_convert_type(fb, jnp.float32)
    return f1 * _USCALE + np.float32(_UMIN - _USCALE)


_NDTRI_POLY = [np.float32(c) for c in (
    -0.00034016495919786394, 0.006403367500752211, -0.047976624220609665,
    0.1778363436460495, -0.3246767818927765, 0.2797957956790924,
    -0.4545946419239044, 0.024007625877857208, -1.2545099258422852)]


def _ndtri_fast(q):
    f32 = np.float32
    s = q * (f32(1.0) - q)
    w = jnp.log2(s) * f32(-np.log(2.0)) + f32(-2.0 * np.log(2.0))
    t = jnp.sqrt(w)
    p = jnp.full_like(q, _NDTRI_POLY[0])
    for c in _NDTRI_POLY[1:]:
        p = p * t + c
    return p * (f32(1.0) - f32(2.0) * q)


BC = 8
NC = 8192


_NLO = np.float32(np.nextafter(np.float32(-1.0), np.float32(0.0), dtype=np.float32))
_NSCALE = np.float32(np.float32(1.0) - _NLO)


_KEPS = (np.uint32(1832780943), np.uint32(270669613))
_KU = (np.uint32(64467757), np.uint32(2916123636))


def _sample_kernel(ms_ref, c_ref, out_ref):
    b = pl.program_id(0)
    n = pl.program_id(1)
    ke1, ke2 = _KEPS
    ku1, ku2 = _KU
    mu = ms_ref[0]
    sigma = ms_ref[1]

    rshape = (BC, NC)
    rb_idx = b * BC + lax.broadcasted_iota(jnp.int32, rshape, 0)
    ri_idx = n * NC + lax.broadcasted_iota(jnp.int32, rshape, 1)
    idx2 = (rb_idx * N + ri_idx).astype(jnp.uint32)
    e0, e1 = _threefry2x32(ke1, ke2, jnp.zeros_like(idx2), idx2)
    ebits = e0 ^ e1
    fb2 = (ebits >> np.uint32(9)) | np.uint32(0x3F800000)
    f2 = lax.bitcast_convert_type(fb2, jnp.float32)
    v = f2 * np.float32(_NSCALE / 2.0) + np.float32((1.0 + _NLO - _NSCALE) / 2.0)
    eps = _ndtri_fast(v)
    upper = mu + sigma * eps
    p_row = np.float32(0.5) * (np.float32(1.0)
                               + lax.erf(upper * np.float32(np.sqrt(0.5))))

    shape = (K, BC, NC)
    k_idx = lax.broadcasted_iota(jnp.int32, shape, 0)
    idx10 = idx2 * np.uint32(K)
    idx = idx10[None, :, :] + k_idx.astype(jnp.uint32)
    o0, o1 = _threefry2x32(ku1, ku2, jnp.zeros_like(idx), idx)
    bits = o0 ^ o1
    u = _bits_to_uniform(bits)

    q = jnp.maximum(u * p_row[None, :, :], np.float32(1e-9))
    z = _ndtri_fast(q)

    c_t = c_ref[...][None, :, :]
    out_ref[...] = jnp.where(c_t == k_idx, upper[None, :, :], z)


def kernel(C, mu, sigma):
    ms = jnp.stack([mu, sigma]).astype(jnp.float32)

    zt = pl.pallas_call(
        _sample_kernel,
        grid=(B // BC, N // NC),
        in_specs=[
            pl.BlockSpec(memory_space=pltpu.SMEM),
            pl.BlockSpec((BC, NC), lambda b, n: (b, n)),
        ],
        out_specs=pl.BlockSpec((K, BC, NC), lambda b, n: (0, b, n)),
        out_shape=jax.ShapeDtypeStruct((K, B, N), jnp.float32),
    )(ms, C)

    return zt.transpose(1, 2, 0)


# speedup vs baseline: 1.0005x; 1.0005x over previous
import numpy as np
import jax
import jax.numpy as jnp
from jax import lax
from jax.experimental import pallas as pl
from jax.experimental.pallas import tpu as pltpu

K = 10
B = 64
N = 8192
_UMIN = np.float32(1e-6)
_UMAX = np.float32(1.0 - 1e-6)
_USCALE = np.float32(_UMAX - _UMIN)


def _threefry2x32(k1, k2, x0, x1):
    rotations = ((13, 15, 26, 6), (17, 29, 16, 24))
    ks0 = k1
    ks1 = k2
    ks2 = k1 ^ k2 ^ np.uint32(0x1BD11BDA)
    ks = (ks0, ks1, ks2)
    v0 = x0 + ks0
    v1 = x1 + ks1
    for i in range(5):
        for r in rotations[i % 2]:
            v0 = v0 + v1
            v1 = (v1 << np.uint32(r)) | (v1 >> np.uint32(32 - r))
            v1 = v0 ^ v1
        v0 = v0 + ks[(i + 1) % 3]
        v1 = v1 + ks[(i + 2) % 3] + np.uint32(i + 1)
    return v0, v1


def _bits_to_uniform(bits):
    fb = (bits >> np.uint32(9)) | np.uint32(0x3F800000)
    f1 = lax.bitcast_convert_type(fb, jnp.float32)
    return f1 * _USCALE + np.float32(_UMIN - _USCALE)


_NDTRI_POLY = [np.float32(c) for c in (
    -0.00034016495919786394, 0.006403367500752211, -0.047976624220609665,
    0.1778363436460495, -0.3246767818927765, 0.2797957956790924,
    -0.4545946419239044, 0.024007625877857208, -1.2545099258422852)]


def _ndtri_fast(q):
    f32 = np.float32
    s = q * (f32(1.0) - q)
    w = jnp.log2(s) * f32(-np.log(2.0)) + f32(-2.0 * np.log(2.0))
    t = jnp.sqrt(w)
    p = jnp.full_like(q, _NDTRI_POLY[0])
    for c in _NDTRI_POLY[1:]:
        p = p * t + c
    return p * (f32(1.0) - f32(2.0) * q)


BC = 8
NC = 8192


_NLO = np.float32(np.nextafter(np.float32(-1.0), np.float32(0.0), dtype=np.float32))
_NSCALE = np.float32(np.float32(1.0) - _NLO)


_KEPS = (np.uint32(1832780943), np.uint32(270669613))
_KU = (np.uint32(64467757), np.uint32(2916123636))


def _sample_kernel(ms_ref, c_ref, out_ref):
    b = pl.program_id(0)
    n = pl.program_id(1)
    ke1, ke2 = _KEPS
    ku1, ku2 = _KU
    mu = ms_ref[0]
    sigma = ms_ref[1]

    rshape = (BC, NC)
    rb_idx = b * BC + lax.broadcasted_iota(jnp.int32, rshape, 0)
    ri_idx = n * NC + lax.broadcasted_iota(jnp.int32, rshape, 1)
    idx2 = (rb_idx * N + ri_idx).astype(jnp.uint32)
    e0, e1 = _threefry2x32(ke1, ke2, np.uint32(0), idx2)
    ebits = e0 ^ e1
    fb2 = (ebits >> np.uint32(9)) | np.uint32(0x3F800000)
    f2 = lax.bitcast_convert_type(fb2, jnp.float32)
    v = f2 * np.float32(_NSCALE / 2.0) + np.float32((1.0 + _NLO - _NSCALE) / 2.0)
    eps = _ndtri_fast(v)
    upper = mu + sigma * eps
    p_row = np.float32(0.5) * (np.float32(1.0)
                               + lax.erf(upper * np.float32(np.sqrt(0.5))))

    shape = (K, BC, NC)
    k_idx = lax.broadcasted_iota(jnp.int32, shape, 0)
    idx10 = idx2 * np.uint32(K)
    idx = idx10[None, :, :] + k_idx.astype(jnp.uint32)
    o0, o1 = _threefry2x32(ku1, ku2, np.uint32(0), idx)
    bits = o0 ^ o1
    u = _bits_to_uniform(bits)

    q = jnp.maximum(u * p_row[None, :, :], np.float32(1e-9))
    z = _ndtri_fast(q)

    c_t = c_ref[...][None, :, :]
    out_ref[...] = jnp.where(c_t == k_idx, upper[None, :, :], z)


def kernel(C, mu, sigma):
    ms = jnp.stack([mu, sigma]).astype(jnp.float32)

    zt = pl.pallas_call(
        _sample_kernel,
        grid=(B // BC, N // NC),
        in_specs=[
            pl.BlockSpec(memory_space=pltpu.SMEM),
            pl.BlockSpec((BC, NC), lambda b, n: (b, n)),
        ],
        out_specs=pl.BlockSpec((K, BC, NC), lambda b, n: (0, b, n)),
        out_shape=jax.ShapeDtypeStruct((K, B, N), jnp.float32),
    )(ms, C)

    return zt.transpose(1, 2, 0)
